# Initial kernel scaffold; baseline (speedup 1.0000x reference)
import jax
import jax.numpy as jnp
from jax.experimental import pallas as pl


def kernel(rpn_probs, rpn_bbox, anchors):
    raise NotImplementedError("write your pallas kernel here")


# speedup vs baseline: 3.2260x; 3.2260x over previous
import jax
import jax.numpy as jnp
from jax.experimental import pallas as pl

_PROPOSAL_COUNT = 2000
_PRE_NMS = 6000
_PAD = 6144
_ROWS = 48
_OROWS = 16
_NMS_THR = 0.7


def _nms_kernel(ag, dg, sc, oy1, ox1, oy2, ox2):
    y1a = ag[0, 0]
    x1a = ag[0, 1]
    y2a = ag[0, 2]
    x2a = ag[0, 3]
    dy = dg[0, 0] * 0.1
    dx = dg[0, 1] * 0.1
    dh = dg[0, 2] * 0.2
    dw = dg[0, 3] * 0.2
    h = y2a - y1a
    w = x2a - x1a
    cy = y1a + 0.5 * h + dy * h
    cx = x1a + 0.5 * w + dx * w
    h = h * jnp.exp(dh)
    w = w * jnp.exp(dw)
    y1u = cy - 0.5 * h
    x1u = cx - 0.5 * w
    y1 = jnp.clip(y1u, 0.0, 1.0)
    x1 = jnp.clip(x1u, 0.0, 1.0)
    y2 = jnp.clip(y1u + h, 0.0, 1.0)
    x2 = jnp.clip(x1u + w, 0.0, 1.0)
    areas = (y2 - y1) * (x2 - x1)

    r = jax.lax.broadcasted_iota(jnp.int32, (_ROWS, 128), 0)
    c = jax.lax.broadcasted_iota(jnp.int32, (_ROWS, 128), 1)
    idxg = r * 128 + c
    ro = jax.lax.broadcasted_iota(jnp.int32, (_OROWS, 128), 0)
    co = jax.lax.broadcasted_iota(jnp.int32, (_OROWS, 128), 1)
    idxo = ro * 128 + co

    zeros_out = jnp.zeros((_OROWS, 128), jnp.float32)
    oy1[0] = zeros_out
    ox1[0] = zeros_out
    oy2[0] = zeros_out
    ox2[0] = zeros_out

    neg_inf = jnp.float32(-jnp.inf)

    def body(k, sv):
        valid = sv > neg_inf
        ok = jnp.any(valid)
        idx = jnp.min(jnp.where(valid, idxg, jnp.int32(0x7FFFFFFF)))
        onehot = idxg == idx
        by1 = jnp.sum(jnp.where(onehot, y1, 0.0))
        bx1 = jnp.sum(jnp.where(onehot, x1, 0.0))
        by2 = jnp.sum(jnp.where(onehot, y2, 0.0))
        bx2 = jnp.sum(jnp.where(onehot, x2, 0.0))
        barea = jnp.sum(jnp.where(onehot, areas, 0.0))
        yy1 = jnp.maximum(y1, by1)
        xx1 = jnp.maximum(x1, bx1)
        yy2 = jnp.minimum(y2, by2)
        xx2 = jnp.minimum(x2, bx2)
        inter = jnp.maximum(yy2 - yy1, 0.0) * jnp.maximum(xx2 - xx1, 0.0)
        iou = inter / (areas + barea - inter + 1e-8)
        supp = (iou > _NMS_THR) | onehot
        sv = jnp.where(supp, neg_inf, sv)
        oh = idxo == k
        oy1[0] = jnp.where(oh, jnp.where(ok, by1, 0.0), oy1[0])
        ox1[0] = jnp.where(oh, jnp.where(ok, bx1, 0.0), ox1[0])
        oy2[0] = jnp.where(oh, jnp.where(ok, by2, 0.0), oy2[0])
        ox2[0] = jnp.where(oh, jnp.where(ok, bx2, 0.0), ox2[0])
        return sv

    jax.lax.fori_loop(0, _PROPOSAL_COUNT, body, sc[0])


def kernel(rpn_probs, rpn_bbox, anchors):
    b = rpn_probs.shape[0]
    scores = rpn_probs[:, :, 1]
    top_scores, ix = jax.lax.top_k(scores, _PRE_NMS)
    deltas_g = jnp.take_along_axis(rpn_bbox, ix[:, :, None], axis=1)
    anchors_g = jnp.take_along_axis(anchors, ix[:, :, None], axis=1)

    pad = _PAD - _PRE_NMS
    sc = jnp.pad(top_scores, ((0, 0), (0, pad)), constant_values=-jnp.inf)
    ag = jnp.pad(anchors_g, ((0, 0), (0, pad), (0, 0)))
    dg = jnp.pad(deltas_g, ((0, 0), (0, pad), (0, 0)))
    ag = ag.transpose(0, 2, 1).reshape(b, 4, _ROWS, 128)
    dg = dg.transpose(0, 2, 1).reshape(b, 4, _ROWS, 128)
    sc = sc.reshape(b, _ROWS, 128)

    out_sds = jax.ShapeDtypeStruct((b, _OROWS, 128), jnp.float32)
    outs = pl.pallas_call(
        _nms_kernel,
        grid=(b,),
        in_specs=[
            pl.BlockSpec((1, 4, _ROWS, 128), lambda i: (i, 0, 0, 0)),
            pl.BlockSpec((1, 4, _ROWS, 128), lambda i: (i, 0, 0, 0)),
            pl.BlockSpec((1, _ROWS, 128), lambda i: (i, 0, 0)),
        ],
        out_specs=[
            pl.BlockSpec((1, _OROWS, 128), lambda i: (i, 0, 0)),
            pl.BlockSpec((1, _OROWS, 128), lambda i: (i, 0, 0)),
            pl.BlockSpec((1, _OROWS, 128), lambda i: (i, 0, 0)),
            pl.BlockSpec((1, _OROWS, 128), lambda i: (i, 0, 0)),
        ],
        out_shape=[out_sds, out_sds, out_sds, out_sds],
    )(ag, dg, sc)

    py1, px1, py2, px2 = [
        o.reshape(b, _OROWS * 128)[:, :_PROPOSAL_COUNT] for o in outs
    ]
    return jnp.stack([py1, px1, py2, px2], axis=-1)
